# Initial kernel scaffold; baseline (speedup 1.0000x reference)
import jax
import jax.numpy as jnp
from jax.experimental import pallas as pl


def kernel(p, x, o, edges, boundary, Wq, bq, Wk, bk, Wv, bv, pW1, pb1, pg, pbeta, pW2, pb2, wg1, wbeta1, wW1, wb1, wg2, wbeta2, wW2, wb2):
    raise NotImplementedError("write your pallas kernel here")


# speedup vs baseline: 3.8530x; 3.8530x over previous
import functools

import jax
import jax.numpy as jnp
from jax import lax
from jax.experimental import pallas as pl
from jax.experimental.pallas import tpu as pltpu
from jax.experimental.pallas import tpu_sc as plsc

N = 50000
K = 16
C = 64
FLAT = N * K
NW = 32
ROWS = 1568
RLAST = N - ROWS
NBT = ROWS * K
CH = 512
NCH = NBT // CH

def _sc_mesh():
    return plsc.VectorSubcoreMesh(core_axis_name="c", subcore_axis_name="s",
                                  num_cores=2, num_subcores=16)


def _idx_body(edges_hbm, bnd_hbm, idx_hbm, bnd_v, edges_v, idx_v):
    w = lax.axis_index("s") * 2 + lax.axis_index("c")
    r0 = jnp.minimum(w * ROWS, RLAST)
    pltpu.sync_copy(bnd_hbm, bnd_v)
    pltpu.sync_copy(edges_hbm.at[pl.ds(r0, ROWS)], edges_v)

    def body(r, carry):
        e = edges_v[r, :]
        bg = plsc.load_gather(bnd_v, [e])
        self_vec = jnp.broadcast_to(r0 + r, (16,)).astype(jnp.int32)
        bi = plsc.load_gather(bnd_v, [self_vec])
        idx_v[r, :] = jnp.where(bg == bi, e, self_vec)
        return carry

    lax.fori_loop(0, ROWS, body, 0)
    pltpu.sync_copy(idx_v, idx_hbm.at[pl.ds(r0, ROWS)])


def _compute_idx(edges, boundary):
    return pl.kernel(
        _idx_body,
        out_type=jax.ShapeDtypeStruct((N, K), jnp.int32),
        mesh=_sc_mesh(),
        compiler_params=pltpu.CompilerParams(needs_layout_passes=False,
                                             use_tc_tiling_on_sc=False),
        scratch_types=[
            pltpu.VMEM((N,), jnp.int32),
            pltpu.VMEM((ROWS, K), jnp.int32),
            pltpu.VMEM((ROWS, K), jnp.int32),
        ],
    )(edges, boundary)


def _gather_body(idx_hbm, xk_hbm, xv_hbm, a_hbm, xkg_hbm, xvg_hbm, ag_hbm,
                 idx_v, xk_b, xv_b, a_b, sk, sv, sa):
    w = lax.axis_index("s") * 2 + lax.axis_index("c")
    r0 = jnp.minimum(w * ROWS, RLAST)
    b0 = r0 * K
    pltpu.sync_copy(idx_hbm.at[pl.ds(b0, NBT)], idx_v)

    def body(c, carry):
        o = c * CH
        ix = idx_v.at[pl.ds(o, CH)]
        cpk = pltpu.async_copy(xk_hbm.at[ix], xk_b, sk)
        cpv = pltpu.async_copy(xv_hbm.at[ix], xv_b, sv)
        cpa = pltpu.async_copy(a_hbm.at[ix], a_b, sa)
        cpk.wait()
        cpv.wait()
        cpa.wait()
        pltpu.sync_copy(xk_b, xkg_hbm.at[pl.ds(b0 + o, CH)])
        pltpu.sync_copy(xv_b, xvg_hbm.at[pl.ds(b0 + o, CH)])
        pltpu.sync_copy(a_b, ag_hbm.at[pl.ds(b0 + o, CH)])
        return carry

    lax.fori_loop(0, NCH, body, 0)


def _gather_rows(idx_flat, xk, xv, a16):
    return pl.kernel(
        _gather_body,
        out_type=(
            jax.ShapeDtypeStruct((FLAT, C), jnp.float32),
            jax.ShapeDtypeStruct((FLAT, C), jnp.float32),
            jax.ShapeDtypeStruct((FLAT, 16), jnp.float32),
        ),
        mesh=_sc_mesh(),
        compiler_params=pltpu.CompilerParams(needs_layout_passes=False,
                                             use_tc_tiling_on_sc=False),
        scratch_types=[
            pltpu.VMEM((NBT,), jnp.int32),
            pltpu.VMEM((CH, C), jnp.float32),
            pltpu.VMEM((CH, C), jnp.float32),
            pltpu.VMEM((CH, 16), jnp.float32),
            pltpu.SemaphoreType.DMA,
            pltpu.SemaphoreType.DMA,
            pltpu.SemaphoreType.DMA,
        ],
    )(idx_flat, xk, xv, a16)


_B1 = 2000
_B = 400


def _qkv_kernel(x_ref, p4_ref, Wq_ref, bq_ref, Wk_ref, bk_ref, Wv_ref, bv_ref,
                pW1p_ref, xq_ref, xk_ref, xv_ref, a_ref):
    x = x_ref[...]
    xq_ref[...] = jnp.dot(x, Wq_ref[...], preferred_element_type=jnp.float32) + bq_ref[...]
    xk_ref[...] = jnp.dot(x, Wk_ref[...], preferred_element_type=jnp.float32) + bk_ref[...]
    xv_ref[...] = jnp.dot(x, Wv_ref[...], preferred_element_type=jnp.float32) + bv_ref[...]
    a_ref[...] = jnp.dot(p4_ref[...], pW1p_ref[...], preferred_element_type=jnp.float32)


def _p_r(ag, aself, s1, t1b, pW2e, pb2):
    d = ag - aself
    h = jnp.maximum(d * s1 + t1b, 0.0)
    b = h.shape[0]
    pr = jnp.dot(h.reshape(b * K, 16), pW2e,
                 preferred_element_type=jnp.float32).reshape(b, K, C)
    return pr + pb2


def _stats1_kernel(ag_ref, aself_ref, out_ref):
    d = ag_ref[...] - aself_ref[...]
    s = jnp.sum(d, axis=(0, 1))
    q = jnp.sum(d * d, axis=(0, 1))

    @pl.when(pl.program_id(0) == 0)
    def _():
        out_ref[...] = jnp.zeros_like(out_ref)

    out_ref[...] = out_ref[...] + jnp.concatenate(
        [s[None], q[None], jnp.zeros((6, 16), jnp.float32)], axis=0)


def _stats2_kernel(xkg_ref, ag_ref, aself_ref, xq_ref, s1_ref, t1b_ref,
                   pW2e_ref, pb2_ref, out_ref):
    pr = _p_r(ag_ref[...], aself_ref[...], s1_ref[...], t1b_ref[...],
              pW2e_ref[...], pb2_ref[...])
    wpre = xkg_ref[...] - xq_ref[...] + pr
    s = jnp.sum(wpre, axis=(0, 1))
    q = jnp.sum(wpre * wpre, axis=(0, 1))

    @pl.when(pl.program_id(0) == 0)
    def _():
        out_ref[...] = jnp.zeros_like(out_ref)

    out_ref[...] = out_ref[...] + jnp.concatenate(
        [s[None], q[None], jnp.zeros((6, C), jnp.float32)], axis=0)


def _w1_kernel(xkg_ref, ag_ref, aself_ref, xq_ref, s1_ref, t1b_ref, pW2e_ref,
               pb2_ref, s2_ref, t2_ref, wW1_ref, wb1_ref, w1_ref, out_ref):
    pr = _p_r(ag_ref[...], aself_ref[...], s1_ref[...], t1b_ref[...],
              pW2e_ref[...], pb2_ref[...])
    wpre = xkg_ref[...] - xq_ref[...] + pr
    h = jnp.maximum(wpre * s2_ref[...] + t2_ref[...], 0.0)
    b = h.shape[0]
    w1 = jnp.dot(h.reshape(b * K, C), wW1_ref[...],
                 preferred_element_type=jnp.float32).reshape(b, K, 8) + wb1_ref[...]
    w1_ref[...] = w1
    s = jnp.sum(w1, axis=(0, 1))
    q = jnp.sum(w1 * w1, axis=(0, 1))

    @pl.when(pl.program_id(0) == 0)
    def _():
        out_ref[...] = jnp.zeros_like(out_ref)

    out_ref[...] = out_ref[...] + jnp.concatenate(
        [s[None], q[None], jnp.zeros((6, 8), jnp.float32)], axis=0)


def _final_kernel(w1_ref, xvg_ref, ag_ref, aself_ref, s1_ref, t1b_ref,
                  pW2e_ref, pb2_ref, s3_ref, t3_ref, wW2_ref, wb2_ref,
                  tile8_ref, out_ref):
    b = w1_ref.shape[0]
    h = jnp.maximum(w1_ref[...] * s3_ref[...] + t3_ref[...], 0.0)
    logit = jnp.dot(h.reshape(b * K, 8), wW2_ref[...],
                    preferred_element_type=jnp.float32).reshape(b, K, 8) + wb2_ref[...]
    m = jnp.max(logit, axis=1, keepdims=True)
    e = jnp.exp(logit - m)
    att = e / jnp.sum(e, axis=1, keepdims=True)
    att_t = jnp.dot(att.reshape(b * K, 8), tile8_ref[...],
                    preferred_element_type=jnp.float32).reshape(b, K, C)
    pr = _p_r(ag_ref[...], aself_ref[...], s1_ref[...], t1b_ref[...],
              pW2e_ref[...], pb2_ref[...])
    v = xvg_ref[...] + pr
    out_ref[...] = jnp.sum(v * att_t, axis=1)


def _const_spec(shape):
    return pl.BlockSpec(shape, lambda i: tuple(0 for _ in shape))


def _moments(stats, nch):
    m = jnp.float32(FLAT)
    mu = stats[0, :nch] / m
    var = jnp.maximum(stats[1, :nch] / m - mu * mu, 0.0)
    return mu, var


def kernel(p, x, o, edges, boundary, Wq, bq, Wk, bk, Wv, bv, pW1, pb1, pg,
           pbeta, pW2, pb2, wg1, wbeta1, wW1, wb1, wg2, wbeta2, wW2, wb2):
    f32 = jnp.float32
    p4 = jnp.pad(p, ((0, 0), (0, 1)))
    pW1p = jnp.zeros((4, 16), f32).at[:3, :3].set(pW1)

    xq, xk, xv, a16 = pl.pallas_call(
        _qkv_kernel,
        grid=(N // _B1,),
        in_specs=[
            pl.BlockSpec((_B1, C), lambda i: (i, 0)),
            pl.BlockSpec((_B1, 4), lambda i: (i, 0)),
            _const_spec((C, C)), _const_spec((1, C)),
            _const_spec((C, C)), _const_spec((1, C)),
            _const_spec((C, C)), _const_spec((1, C)),
            _const_spec((4, 16)),
        ],
        out_specs=[
            pl.BlockSpec((_B1, C), lambda i: (i, 0)),
            pl.BlockSpec((_B1, C), lambda i: (i, 0)),
            pl.BlockSpec((_B1, C), lambda i: (i, 0)),
            pl.BlockSpec((_B1, 16), lambda i: (i, 0)),
        ],
        out_shape=[
            jax.ShapeDtypeStruct((N, C), f32),
            jax.ShapeDtypeStruct((N, C), f32),
            jax.ShapeDtypeStruct((N, C), f32),
            jax.ShapeDtypeStruct((N, 16), f32),
        ],
    )(x, p4, Wq, bq.reshape(1, C), Wk, bk.reshape(1, C), Wv, bv.reshape(1, C),
      pW1p)

    idx = _compute_idx(edges.astype(jnp.int32), boundary.astype(jnp.int32))

    xkg, xvg, ag = _gather_rows(idx.reshape(FLAT), xk, xv, a16)
    xkg3 = xkg.reshape(N, K, C)
    xvg3 = xvg.reshape(N, K, C)
    ag3 = ag.reshape(N, K, 16)
    aself = a16.reshape(N, 1, 16)
    xq3 = xq.reshape(N, 1, C)

    grid = (N // _B,)

    st1 = pl.pallas_call(
        _stats1_kernel,
        grid=grid,
        in_specs=[
            pl.BlockSpec((_B, K, 16), lambda i: (i, 0, 0)),
            pl.BlockSpec((_B, 1, 16), lambda i: (i, 0, 0)),
        ],
        out_specs=pl.BlockSpec((8, 16), lambda i: (0, 0)),
        out_shape=jax.ShapeDtypeStruct((8, 16), f32),
    )(ag3, aself)
    mu1, var1 = _moments(st1, 16)
    lane3 = jnp.arange(16) < 3
    pg16 = jnp.zeros((16,), f32).at[:3].set(pg)
    pbeta16 = jnp.zeros((16,), f32).at[:3].set(pbeta)
    s1 = jnp.where(lane3, pg16 / jnp.sqrt(var1 + 1e-5), 0.0)
    t1b = jnp.where(lane3, pbeta16 - mu1 * s1, 0.0)
    s1c = s1.reshape(1, 1, 16)
    t1c = t1b.reshape(1, 1, 16)
    pW2e = jnp.zeros((16, C), f32).at[:3, :].set(pW2)
    pb2c = pb2.reshape(1, 1, C)

    st2 = pl.pallas_call(
        _stats2_kernel,
        grid=grid,
        in_specs=[
            pl.BlockSpec((_B, K, C), lambda i: (i, 0, 0)),
            pl.BlockSpec((_B, K, 16), lambda i: (i, 0, 0)),
            pl.BlockSpec((_B, 1, 16), lambda i: (i, 0, 0)),
            pl.BlockSpec((_B, 1, C), lambda i: (i, 0, 0)),
            _const_spec((1, 1, 16)), _const_spec((1, 1, 16)),
            _const_spec((16, C)), _const_spec((1, 1, C)),
        ],
        out_specs=pl.BlockSpec((8, C), lambda i: (0, 0)),
        out_shape=jax.ShapeDtypeStruct((8, C), f32),
    )(xkg3, ag3, aself, xq3, s1c, t1c, pW2e, pb2c)
    mu2, var2 = _moments(st2, C)
    s2 = wg1 / jnp.sqrt(var2 + 1e-5)
    t2 = wbeta1 - mu2 * s2

    w1, st3 = pl.pallas_call(
        _w1_kernel,
        grid=grid,
        in_specs=[
            pl.BlockSpec((_B, K, C), lambda i: (i, 0, 0)),
            pl.BlockSpec((_B, K, 16), lambda i: (i, 0, 0)),
            pl.BlockSpec((_B, 1, 16), lambda i: (i, 0, 0)),
            pl.BlockSpec((_B, 1, C), lambda i: (i, 0, 0)),
            _const_spec((1, 1, 16)), _const_spec((1, 1, 16)),
            _const_spec((16, C)), _const_spec((1, 1, C)),
            _const_spec((1, 1, C)), _const_spec((1, 1, C)),
            _const_spec((C, 8)), _const_spec((1, 1, 8)),
        ],
        out_specs=[
            pl.BlockSpec((_B, K, 8), lambda i: (i, 0, 0)),
            pl.BlockSpec((8, 8), lambda i: (0, 0)),
        ],
        out_shape=[
            jax.ShapeDtypeStruct((N, K, 8), f32),
            jax.ShapeDtypeStruct((8, 8), f32),
        ],
    )(xkg3, ag3, aself, xq3, s1c, t1c, pW2e, pb2c,
      s2.reshape(1, 1, C), t2.reshape(1, 1, C), wW1, wb1.reshape(1, 1, 8))
    mu3, var3 = _moments(st3, 8)
    s3 = wg2 / jnp.sqrt(var3 + 1e-5)
    t3 = wbeta2 - mu3 * s3

    tile8 = jnp.tile(jnp.eye(8, dtype=f32), (1, 8))
    out = pl.pallas_call(
        _final_kernel,
        grid=grid,
        in_specs=[
            pl.BlockSpec((_B, K, 8), lambda i: (i, 0, 0)),
            pl.BlockSpec((_B, K, C), lambda i: (i, 0, 0)),
            pl.BlockSpec((_B, K, 16), lambda i: (i, 0, 0)),
            pl.BlockSpec((_B, 1, 16), lambda i: (i, 0, 0)),
            _const_spec((1, 1, 16)), _const_spec((1, 1, 16)),
            _const_spec((16, C)), _const_spec((1, 1, C)),
            _const_spec((1, 1, 8)), _const_spec((1, 1, 8)),
            _const_spec((8, 8)), _const_spec((1, 1, 8)),
            _const_spec((8, C)),
        ],
        out_specs=pl.BlockSpec((_B, C), lambda i: (i, 0)),
        out_shape=jax.ShapeDtypeStruct((N, C), f32),
    )(w1, xvg3, ag3, aself, s1c, t1c, pW2e, pb2c,
      s3.reshape(1, 1, 8), t3.reshape(1, 1, 8), wW2, wb2.reshape(1, 1, 8),
      tile8)
    return out


# speedup vs baseline: 6.9358x; 1.8001x over previous
import functools

import jax
import jax.numpy as jnp
from jax import lax
from jax.experimental import pallas as pl
from jax.experimental.pallas import tpu as pltpu
from jax.experimental.pallas import tpu_sc as plsc

N = 50000
K = 16
C = 64
FLAT = N * K
NW = 32
ROWS = 1568
RLAST = N - ROWS
NBT = ROWS * K
CH = 512
NCH = NBT // CH


def _sc_mesh():
    return plsc.VectorSubcoreMesh(core_axis_name="c", subcore_axis_name="s",
                                  num_cores=2, num_subcores=16)


_SC_PARAMS = pltpu.CompilerParams(needs_layout_passes=False,
                                  use_tc_tiling_on_sc=False)


def _idx_body(edges_hbm, bnd_hbm, idx_hbm, bnd_v, edges_v, idx_v):
    w = lax.axis_index("s") * 2 + lax.axis_index("c")
    r0 = jnp.minimum(w * ROWS, RLAST)
    pltpu.sync_copy(bnd_hbm, bnd_v)
    pltpu.sync_copy(edges_hbm.at[pl.ds(r0, ROWS)], edges_v)

    def body(r, carry):
        e = edges_v[r, :]
        bg = plsc.load_gather(bnd_v, [e])
        self_vec = jnp.broadcast_to(r0 + r, (16,)).astype(jnp.int32)
        bi = plsc.load_gather(bnd_v, [self_vec])
        idx_v[r, :] = jnp.where(bg == bi, e, self_vec)
        return carry

    lax.fori_loop(0, ROWS, body, 0)
    pltpu.sync_copy(idx_v, idx_hbm.at[pl.ds(r0, ROWS)])


def _compute_idx(edges, boundary):
    return pl.kernel(
        _idx_body,
        out_type=jax.ShapeDtypeStruct((N, K), jnp.int32),
        mesh=_sc_mesh(),
        compiler_params=_SC_PARAMS,
        scratch_types=[
            pltpu.VMEM((N,), jnp.int32),
            pltpu.VMEM((ROWS, K), jnp.int32),
            pltpu.VMEM((ROWS, K), jnp.int32),
        ],
    )(edges, boundary)


def _gather_body(idx_hbm, xk_hbm, xv_hbm, a_hbm, xkg_hbm, xvg_hbm, ag_hbm,
                 idx_v, xk_b, xv_b, a_b, sk, sv, sa):
    w = lax.axis_index("s") * 2 + lax.axis_index("c")
    r0 = jnp.minimum(w * ROWS, RLAST)
    b0 = r0 * K
    pltpu.sync_copy(idx_hbm.at[pl.ds(b0, NBT)], idx_v)

    def body(c, carry):
        o = c * CH
        ix = idx_v.at[pl.ds(o, CH)]
        cpk = pltpu.async_copy(xk_hbm.at[ix], xk_b, sk)
        cpv = pltpu.async_copy(xv_hbm.at[ix], xv_b, sv)
        cpa = pltpu.async_copy(a_hbm.at[ix], a_b, sa)
        cpk.wait()
        cpv.wait()
        cpa.wait()
        pltpu.sync_copy(xk_b, xkg_hbm.at[pl.ds(b0 + o, CH)])
        pltpu.sync_copy(xv_b, xvg_hbm.at[pl.ds(b0 + o, CH)])
        pltpu.sync_copy(a_b, ag_hbm.at[pl.ds(b0 + o, CH)])
        return carry

    lax.fori_loop(0, NCH, body, 0)


def _gather_rows(idx_flat, xk, xv, a16):
    return pl.kernel(
        _gather_body,
        out_type=(
            jax.ShapeDtypeStruct((FLAT, C), jnp.float32),
            jax.ShapeDtypeStruct((FLAT, C), jnp.float32),
            jax.ShapeDtypeStruct((FLAT, 16), jnp.float32),
        ),
        mesh=_sc_mesh(),
        compiler_params=_SC_PARAMS,
        scratch_types=[
            pltpu.VMEM((NBT,), jnp.int32),
            pltpu.VMEM((CH, C), jnp.float32),
            pltpu.VMEM((CH, C), jnp.float32),
            pltpu.VMEM((CH, 16), jnp.float32),
            pltpu.SemaphoreType.DMA,
            pltpu.SemaphoreType.DMA,
            pltpu.SemaphoreType.DMA,
        ],
    )(idx_flat, xk, xv, a16)


_B1 = 2000
_B = 1000


def _qkv_kernel(x_ref, p4_ref, Wq_ref, bq_ref, Wk_ref, bk_ref, Wv_ref, bv_ref,
                pW1p_ref, xq_ref, xk_ref, xv_ref, a_ref):
    x = x_ref[...]
    xq_ref[...] = jnp.dot(x, Wq_ref[...], preferred_element_type=jnp.float32) + bq_ref[...]
    xk_ref[...] = jnp.dot(x, Wk_ref[...], preferred_element_type=jnp.float32) + bk_ref[...]
    xv_ref[...] = jnp.dot(x, Wv_ref[...], preferred_element_type=jnp.float32) + bv_ref[...]
    a_ref[...] = jnp.dot(p4_ref[...], pW1p_ref[...], preferred_element_type=jnp.float32)


def _mm(a, b):
    return jnp.dot(a, b, preferred_element_type=jnp.float32)


def _p_r(ag, a16, t16, s1, t1b, wbd2, pb2t):
    d = ag - _mm(a16, t16)
    h = jnp.maximum(d * s1 + t1b, 0.0)
    return _mm(h, wbd2) + pb2t


def _acc_stats(out_ref, s, q, width):
    @pl.when(pl.program_id(0) == 0)
    def _():
        out_ref[...] = jnp.zeros_like(out_ref)

    out_ref[...] = out_ref[...] + jnp.concatenate(
        [s[None], q[None], jnp.zeros((6, width), jnp.float32)], axis=0)


def _stats1_kernel(ag_ref, a16_ref, t16_ref, out_ref):
    d = ag_ref[...] - _mm(a16_ref[...], t16_ref[...])
    _acc_stats(out_ref, jnp.sum(d, axis=0), jnp.sum(d * d, axis=0), 256)


def _stats2_kernel(xkg_ref, ag_ref, a16_ref, xq_ref, t16_ref, t64_ref,
                   s1_ref, t1b_ref, wbd2_ref, pb2_ref, out_ref):
    pr = _p_r(ag_ref[...], a16_ref[...], t16_ref[...], s1_ref[...],
              t1b_ref[...], wbd2_ref[...], pb2_ref[...])
    wpre = xkg_ref[...] - _mm(xq_ref[...], t64_ref[...]) + pr
    _acc_stats(out_ref, jnp.sum(wpre, axis=0), jnp.sum(wpre * wpre, axis=0),
               16 * C)


def _w1_kernel(xkg_ref, ag_ref, a16_ref, xq_ref, t16_ref, t64_ref, s1_ref,
               t1b_ref, wbd2_ref, pb2_ref, s2_ref, t2_ref, wbd1_ref, wb1_ref,
               w1_ref, out_ref):
    pr = _p_r(ag_ref[...], a16_ref[...], t16_ref[...], s1_ref[...],
              t1b_ref[...], wbd2_ref[...], pb2_ref[...])
    wpre = xkg_ref[...] - _mm(xq_ref[...], t64_ref[...]) + pr
    h = jnp.maximum(wpre * s2_ref[...] + t2_ref[...], 0.0)
    w1 = _mm(h, wbd1_ref[...]) + wb1_ref[...]
    w1_ref[...] = w1
    _acc_stats(out_ref, jnp.sum(w1, axis=0), jnp.sum(w1 * w1, axis=0), 128)


def _final_kernel(w1_ref, xvg_ref, ag_ref, a16_ref, t16_ref, s1_ref, t1b_ref,
                  wbd2_ref, pb2_ref, s3_ref, t3_ref, wbd0_ref, wb2_ref,
                  msum_ref, expand_ref, reduce_ref, out_ref):
    h = jnp.maximum(w1_ref[...] * s3_ref[...] + t3_ref[...], 0.0)
    logit = _mm(h, wbd0_ref[...]) + wb2_ref[...]
    e = jnp.exp(logit)
    se = _mm(e, msum_ref[...])
    att = e / se
    attx = _mm(att, expand_ref[...])
    pr = _p_r(ag_ref[...], a16_ref[...], t16_ref[...], s1_ref[...],
              t1b_ref[...], wbd2_ref[...], pb2_ref[...])
    v = xvg_ref[...] + pr
    out_ref[...] = _mm(v * attx, reduce_ref[...])


def _const_spec(shape):
    return pl.BlockSpec(shape, lambda i: tuple(0 for _ in shape))


def _moments(stats, nch):
    m = jnp.float32(FLAT)
    mu = stats[0].reshape(K, nch).sum(0) / m
    var = jnp.maximum(stats[1].reshape(K, nch).sum(0) / m - mu * mu, 0.0)
    return mu, var


def kernel(p, x, o, edges, boundary, Wq, bq, Wk, bk, Wv, bv, pW1, pb1, pg,
           pbeta, pW2, pb2, wg1, wbeta1, wW1, wb1, wg2, wbeta2, wW2, wb2):
    f32 = jnp.float32
    p4 = jnp.pad(p, ((0, 0), (0, 1)))
    pW1p = jnp.zeros((4, 16), f32).at[:3, :3].set(pW1)

    xq, xk, xv, a16 = pl.pallas_call(
        _qkv_kernel,
        grid=(N // _B1,),
        in_specs=[
            pl.BlockSpec((_B1, C), lambda i: (i, 0)),
            pl.BlockSpec((_B1, 4), lambda i: (i, 0)),
            _const_spec((C, C)), _const_spec((1, C)),
            _const_spec((C, C)), _const_spec((1, C)),
            _const_spec((C, C)), _const_spec((1, C)),
            _const_spec((4, 16)),
        ],
        out_specs=[
            pl.BlockSpec((_B1, C), lambda i: (i, 0)),
            pl.BlockSpec((_B1, C), lambda i: (i, 0)),
            pl.BlockSpec((_B1, C), lambda i: (i, 0)),
            pl.BlockSpec((_B1, 16), lambda i: (i, 0)),
        ],
        out_shape=[
            jax.ShapeDtypeStruct((N, C), f32),
            jax.ShapeDtypeStruct((N, C), f32),
            jax.ShapeDtypeStruct((N, C), f32),
            jax.ShapeDtypeStruct((N, 16), f32),
        ],
    )(x, p4, Wq, bq.reshape(1, C), Wk, bk.reshape(1, C), Wv, bv.reshape(1, C),
      pW1p)

    idx = _compute_idx(edges.astype(jnp.int32), boundary.astype(jnp.int32))

    xkg, xvg, ag = _gather_rows(idx.reshape(FLAT), xk, xv, a16)
    xkg2 = xkg.reshape(N, K * C)
    xvg2 = xvg.reshape(N, K * C)
    ag2 = ag.reshape(N, K * 16)

    eye16 = jnp.eye(16, dtype=f32)
    eye64 = jnp.eye(C, dtype=f32)
    eye8 = jnp.eye(8, dtype=f32)
    t16 = jnp.tile(eye16, (1, K))
    t64 = jnp.tile(eye64, (1, K))
    pW2e = jnp.zeros((16, C), f32).at[:3, :].set(pW2)
    wbd2 = jnp.kron(eye16, pW2e)
    msum = jnp.tile(eye8, (K, K))
    expand = jnp.kron(eye16, jnp.tile(eye8, (1, 8)))
    reduce_m = jnp.tile(eye64, (K, 1))

    grid = (N // _B,)

    st1 = pl.pallas_call(
        _stats1_kernel,
        grid=grid,
        in_specs=[
            pl.BlockSpec((_B, K * 16), lambda i: (i, 0)),
            pl.BlockSpec((_B, 16), lambda i: (i, 0)),
            _const_spec((16, K * 16)),
        ],
        out_specs=pl.BlockSpec((8, K * 16), lambda i: (0, 0)),
        out_shape=jax.ShapeDtypeStruct((8, K * 16), f32),
    )(ag2, a16, t16)
    mu1, var1 = _moments(st1, 16)
    lane3 = jnp.arange(16) < 3
    pg16 = jnp.zeros((16,), f32).at[:3].set(pg)
    pbeta16 = jnp.zeros((16,), f32).at[:3].set(pbeta)
    s1 = jnp.where(lane3, pg16 / jnp.sqrt(var1 + 1e-5), 0.0)
    t1b = jnp.where(lane3, pbeta16 - mu1 * s1, 0.0)
    s1c = jnp.tile(s1, K).reshape(1, K * 16)
    t1c = jnp.tile(t1b, K).reshape(1, K * 16)
    pb2t = jnp.tile(pb2, K).reshape(1, K * C)

    st2 = pl.pallas_call(
        _stats2_kernel,
        grid=grid,
        in_specs=[
            pl.BlockSpec((_B, K * C), lambda i: (i, 0)),
            pl.BlockSpec((_B, K * 16), lambda i: (i, 0)),
            pl.BlockSpec((_B, 16), lambda i: (i, 0)),
            pl.BlockSpec((_B, C), lambda i: (i, 0)),
            _const_spec((16, K * 16)), _const_spec((C, K * C)),
            _const_spec((1, K * 16)), _const_spec((1, K * 16)),
            _const_spec((K * 16, K * C)), _const_spec((1, K * C)),
        ],
        out_specs=pl.BlockSpec((8, K * C), lambda i: (0, 0)),
        out_shape=jax.ShapeDtypeStruct((8, K * C), f32),
    )(xkg2, ag2, a16, xq, t16, t64, s1c, t1c, wbd2, pb2t)
    mu2, var2 = _moments(st2, C)
    s2 = wg1 / jnp.sqrt(var2 + 1e-5)
    t2 = wbeta1 - mu2 * s2

    wbd1 = jnp.kron(eye16, wW1)
    w1, st3 = pl.pallas_call(
        _w1_kernel,
        grid=grid,
        in_specs=[
            pl.BlockSpec((_B, K * C), lambda i: (i, 0)),
            pl.BlockSpec((_B, K * 16), lambda i: (i, 0)),
            pl.BlockSpec((_B, 16), lambda i: (i, 0)),
            pl.BlockSpec((_B, C), lambda i: (i, 0)),
            _const_spec((16, K * 16)), _const_spec((C, K * C)),
            _const_spec((1, K * 16)), _const_spec((1, K * 16)),
            _const_spec((K * 16, K * C)), _const_spec((1, K * C)),
            _const_spec((1, K * C)), _const_spec((1, K * C)),
            _const_spec((K * C, 128)), _const_spec((1, 128)),
        ],
        out_specs=[
            pl.BlockSpec((_B, 128), lambda i: (i, 0)),
            pl.BlockSpec((8, 128), lambda i: (0, 0)),
        ],
        out_shape=[
            jax.ShapeDtypeStruct((N, 128), f32),
            jax.ShapeDtypeStruct((8, 128), f32),
        ],
    )(xkg2, ag2, a16, xq, t16, t64, s1c, t1c, wbd2, pb2t,
      jnp.tile(s2, K).reshape(1, K * C), jnp.tile(t2, K).reshape(1, K * C),
      wbd1, jnp.tile(wb1, K).reshape(1, 128))
    mu3, var3 = _moments(st3, 8)
    s3 = wg2 / jnp.sqrt(var3 + 1e-5)
    t3 = wbeta2 - mu3 * s3

    wbd0 = jnp.kron(eye16, wW2)
    out = pl.pallas_call(
        _final_kernel,
        grid=grid,
        in_specs=[
            pl.BlockSpec((_B, 128), lambda i: (i, 0)),
            pl.BlockSpec((_B, K * C), lambda i: (i, 0)),
            pl.BlockSpec((_B, K * 16), lambda i: (i, 0)),
            pl.BlockSpec((_B, 16), lambda i: (i, 0)),
            _const_spec((16, K * 16)),
            _const_spec((1, K * 16)), _const_spec((1, K * 16)),
            _const_spec((K * 16, K * C)), _const_spec((1, K * C)),
            _const_spec((1, 128)), _const_spec((1, 128)),
            _const_spec((128, 128)), _const_spec((1, 128)),
            _const_spec((128, 128)), _const_spec((128, K * C)),
            _const_spec((K * C, C)),
        ],
        out_specs=pl.BlockSpec((_B, C), lambda i: (i, 0)),
        out_shape=jax.ShapeDtypeStruct((N, C), f32),
    )(w1, xvg2, ag2, a16, t16, s1c, t1c, wbd2, pb2t,
      jnp.tile(s3, K).reshape(1, 128), jnp.tile(t3, K).reshape(1, 128),
      wbd0, jnp.tile(wb2, K).reshape(1, 128), msum, expand, reduce_m)
    return out


# speedup vs baseline: 7.3609x; 1.0613x over previous
import functools

import jax
import jax.numpy as jnp
from jax import lax
from jax.experimental import pallas as pl
from jax.experimental.pallas import tpu as pltpu
from jax.experimental.pallas import tpu_sc as plsc

N = 50000
K = 16
C = 64
FLAT = N * K
NW = 32
ROWS = 1568
RLAST = N - ROWS
NBT = ROWS * K


def _sc_mesh():
    return plsc.VectorSubcoreMesh(core_axis_name="c", subcore_axis_name="s",
                                  num_cores=2, num_subcores=16)


_SC_PARAMS = pltpu.CompilerParams(needs_layout_passes=False,
                                  use_tc_tiling_on_sc=False)


def _idx_body(edges_hbm, bnd_hbm, idx_hbm, bnd_v, edges_v, idx_v):
    w = lax.axis_index("s") * 2 + lax.axis_index("c")
    r0 = jnp.minimum(w * ROWS, RLAST)
    pltpu.sync_copy(bnd_hbm, bnd_v)
    pltpu.sync_copy(edges_hbm.at[pl.ds(r0, ROWS)], edges_v)

    def body(r, carry):
        e = edges_v[r, :]
        bg = plsc.load_gather(bnd_v, [e])
        self_vec = jnp.broadcast_to(r0 + r, (16,)).astype(jnp.int32)
        bi = plsc.load_gather(bnd_v, [self_vec])
        idx_v[r, :] = jnp.where(bg == bi, e, self_vec)
        return carry

    lax.fori_loop(0, ROWS, body, 0)
    pltpu.sync_copy(idx_v, idx_hbm.at[pl.ds(r0, ROWS)])


def _compute_idx(edges, boundary):
    return pl.kernel(
        _idx_body,
        out_type=jax.ShapeDtypeStruct((N, K), jnp.int32),
        mesh=_sc_mesh(),
        compiler_params=_SC_PARAMS,
        scratch_types=[
            pltpu.VMEM((N,), jnp.int32),
            pltpu.VMEM((ROWS, K), jnp.int32),
            pltpu.VMEM((ROWS, K), jnp.int32),
        ],
    )(edges, boundary)


GCH = 448
GNCH = NBT // GCH
GPAIRS = GNCH // 2


def _make_gather_body(widths):
    nt = len(widths)

    def body(*refs):
        idx_hbm = refs[0]
        tables = refs[1:1 + nt]
        outs = refs[1 + nt:1 + 2 * nt]
        idx_v = refs[1 + 2 * nt]
        bufs = refs[2 + 2 * nt:2 + 4 * nt]
        sems = refs[2 + 4 * nt:2 + 6 * nt]
        w = lax.axis_index("s") * 2 + lax.axis_index("c")
        r0 = jnp.minimum(w * ROWS, RLAST)
        b0 = r0 * K
        pltpu.sync_copy(idx_hbm.at[pl.ds(b0, NBT)], idx_v)

        def issue(c, par):
            ix = idx_v.at[pl.ds(c * GCH, GCH)]
            for t in range(nt):
                pltpu.async_copy(tables[t].at[ix], bufs[2 * t + par],
                                 sems[2 * t + par])

        def wait(c, par):
            ix = idx_v.at[pl.ds(c * GCH, GCH)]
            for t in range(nt):
                pltpu.make_async_copy(tables[t].at[ix], bufs[2 * t + par],
                                      sems[2 * t + par]).wait()

        def store(c, par):
            for t in range(nt):
                pltpu.sync_copy(bufs[2 * t + par],
                                outs[t].at[pl.ds(b0 + c * GCH, GCH)])

        issue(0, 0)

        def pair(i, carry):
            c0 = 2 * i
            issue(c0 + 1, 1)
            wait(c0, 0)
            store(c0, 0)

            @pl.when(i < GPAIRS - 1)
            def _():
                issue(c0 + 2, 0)

            wait(c0 + 1, 1)
            store(c0 + 1, 1)
            return carry

        lax.fori_loop(0, GPAIRS, pair, 0)

    return body


def _gather_multi(idx_flat, tables, widths):
    return pl.kernel(
        _make_gather_body(widths),
        out_type=tuple(jax.ShapeDtypeStruct((FLAT, wd), jnp.float32)
                       for wd in widths),
        mesh=_sc_mesh(),
        compiler_params=_SC_PARAMS,
        scratch_types=[pltpu.VMEM((NBT,), jnp.int32)]
        + [pltpu.VMEM((GCH, wd), jnp.float32)
           for wd in widths for _ in range(2)]
        + [pltpu.SemaphoreType.DMA for _ in widths for _ in range(2)],
    )(idx_flat, *tables)


_B1 = 2000
_B = 1000


def _qkv_kernel(x_ref, p4_ref, Wq_ref, bq_ref, Wk_ref, bk_ref, Wv_ref, bv_ref,
                pW1p_ref, xq_ref, xk_ref, xv_ref, a_ref):
    x = x_ref[...]
    xq_ref[...] = jnp.dot(x, Wq_ref[...], preferred_element_type=jnp.float32) + bq_ref[...]
    xk_ref[...] = jnp.dot(x, Wk_ref[...], preferred_element_type=jnp.float32) + bk_ref[...]
    xv_ref[...] = jnp.dot(x, Wv_ref[...], preferred_element_type=jnp.float32) + bv_ref[...]
    a_ref[...] = jnp.dot(p4_ref[...], pW1p_ref[...], preferred_element_type=jnp.float32)


def _mm(a, b):
    return jnp.dot(a, b, preferred_element_type=jnp.float32)


def _p_r(ag, a16, t16, s1, t1b, wbd2, pb2t):
    d = ag - _mm(a16, t16)
    h = jnp.maximum(d * s1 + t1b, 0.0)
    return _mm(h, wbd2) + pb2t


def _acc_stats(out_ref, s, q, width):
    @pl.when(pl.program_id(0) == 0)
    def _():
        out_ref[...] = jnp.zeros_like(out_ref)

    out_ref[...] = out_ref[...] + jnp.concatenate(
        [s[None], q[None], jnp.zeros((6, width), jnp.float32)], axis=0)


def _stats1_kernel(ag_ref, a16_ref, t16_ref, out_ref):
    d = ag_ref[...] - _mm(a16_ref[...], t16_ref[...])
    _acc_stats(out_ref, jnp.sum(d, axis=0), jnp.sum(d * d, axis=0), 256)


def _stats2_kernel(xkg_ref, ag_ref, a16_ref, xq_ref, t16_ref, t64_ref,
                   s1_ref, t1b_ref, wbd2_ref, pb2_ref, out_ref):
    pr = _p_r(ag_ref[...], a16_ref[...], t16_ref[...], s1_ref[...],
              t1b_ref[...], wbd2_ref[...], pb2_ref[...])
    wpre = xkg_ref[...] - _mm(xq_ref[...], t64_ref[...]) + pr
    _acc_stats(out_ref, jnp.sum(wpre, axis=0), jnp.sum(wpre * wpre, axis=0),
               16 * C)


def _w1_kernel(xkg_ref, ag_ref, a16_ref, xq_ref, t16_ref, t64_ref, s1_ref,
               t1b_ref, wbd2_ref, pb2_ref, s2_ref, t2_ref, wbd1_ref, wb1_ref,
               w1_ref, out_ref):
    pr = _p_r(ag_ref[...], a16_ref[...], t16_ref[...], s1_ref[...],
              t1b_ref[...], wbd2_ref[...], pb2_ref[...])
    wpre = xkg_ref[...] - _mm(xq_ref[...], t64_ref[...]) + pr
    h = jnp.maximum(wpre * s2_ref[...] + t2_ref[...], 0.0)
    w1 = _mm(h, wbd1_ref[...]) + wb1_ref[...]
    w1_ref[...] = w1
    _acc_stats(out_ref, jnp.sum(w1, axis=0), jnp.sum(w1 * w1, axis=0), 128)


def _final_kernel(w1_ref, xvg_ref, ag_ref, a16_ref, t16_ref, s1_ref, t1b_ref,
                  wbd2_ref, pb2_ref, s3_ref, t3_ref, wbd0_ref, wb2_ref,
                  msum_ref, expand_ref, reduce_ref, out_ref):
    h = jnp.maximum(w1_ref[...] * s3_ref[...] + t3_ref[...], 0.0)
    logit = _mm(h, wbd0_ref[...]) + wb2_ref[...]
    e = jnp.exp(logit)
    se = _mm(e, msum_ref[...])
    att = e / se
    attx = _mm(att, expand_ref[...])
    pr = _p_r(ag_ref[...], a16_ref[...], t16_ref[...], s1_ref[...],
              t1b_ref[...], wbd2_ref[...], pb2_ref[...])
    v = xvg_ref[...] + pr
    out_ref[...] = _mm(v * attx, reduce_ref[...])


def _const_spec(shape):
    return pl.BlockSpec(shape, lambda i: tuple(0 for _ in shape))


def _moments(stats, nch):
    m = jnp.float32(FLAT)
    mu = stats[0].reshape(K, nch).sum(0) / m
    var = jnp.maximum(stats[1].reshape(K, nch).sum(0) / m - mu * mu, 0.0)
    return mu, var


def kernel(p, x, o, edges, boundary, Wq, bq, Wk, bk, Wv, bv, pW1, pb1, pg,
           pbeta, pW2, pb2, wg1, wbeta1, wW1, wb1, wg2, wbeta2, wW2, wb2):
    f32 = jnp.float32
    p4 = jnp.pad(p, ((0, 0), (0, 1)))
    pW1p = jnp.zeros((4, 16), f32).at[:3, :3].set(pW1)

    xq, xk, xv, a16 = pl.pallas_call(
        _qkv_kernel,
        grid=(N // _B1,),
        in_specs=[
            pl.BlockSpec((_B1, C), lambda i: (i, 0)),
            pl.BlockSpec((_B1, 4), lambda i: (i, 0)),
            _const_spec((C, C)), _const_spec((1, C)),
            _const_spec((C, C)), _const_spec((1, C)),
            _const_spec((C, C)), _const_spec((1, C)),
            _const_spec((4, 16)),
        ],
        out_specs=[
            pl.BlockSpec((_B1, C), lambda i: (i, 0)),
            pl.BlockSpec((_B1, C), lambda i: (i, 0)),
            pl.BlockSpec((_B1, C), lambda i: (i, 0)),
            pl.BlockSpec((_B1, 16), lambda i: (i, 0)),
        ],
        out_shape=[
            jax.ShapeDtypeStruct((N, C), f32),
            jax.ShapeDtypeStruct((N, C), f32),
            jax.ShapeDtypeStruct((N, C), f32),
            jax.ShapeDtypeStruct((N, 16), f32),
        ],
    )(x, p4, Wq, bq.reshape(1, C), Wk, bk.reshape(1, C), Wv, bv.reshape(1, C),
      pW1p)

    idx = _compute_idx(edges.astype(jnp.int32), boundary.astype(jnp.int32))

    idx_flat = idx.reshape(FLAT)
    xkg, ag = _gather_multi(idx_flat, (xk, a16), (C, 16))
    (xvg,) = _gather_multi(idx_flat, (xv,), (C,))
    xkg2 = xkg.reshape(N, K * C)
    xvg2 = xvg.reshape(N, K * C)
    ag2 = ag.reshape(N, K * 16)

    eye16 = jnp.eye(16, dtype=f32)
    eye64 = jnp.eye(C, dtype=f32)
    eye8 = jnp.eye(8, dtype=f32)
    t16 = jnp.tile(eye16, (1, K))
    t64 = jnp.tile(eye64, (1, K))
    pW2e = jnp.zeros((16, C), f32).at[:3, :].set(pW2)
    wbd2 = jnp.kron(eye16, pW2e)
    msum = jnp.tile(eye8, (K, K))
    expand = jnp.kron(eye16, jnp.tile(eye8, (1, 8)))
    reduce_m = jnp.tile(eye64, (K, 1))

    grid = (N // _B,)

    st1 = pl.pallas_call(
        _stats1_kernel,
        grid=grid,
        in_specs=[
            pl.BlockSpec((_B, K * 16), lambda i: (i, 0)),
            pl.BlockSpec((_B, 16), lambda i: (i, 0)),
            _const_spec((16, K * 16)),
        ],
        out_specs=pl.BlockSpec((8, K * 16), lambda i: (0, 0)),
        out_shape=jax.ShapeDtypeStruct((8, K * 16), f32),
    )(ag2, a16, t16)
    mu1, var1 = _moments(st1, 16)
    lane3 = jnp.arange(16) < 3
    pg16 = jnp.zeros((16,), f32).at[:3].set(pg)
    pbeta16 = jnp.zeros((16,), f32).at[:3].set(pbeta)
    s1 = jnp.where(lane3, pg16 / jnp.sqrt(var1 + 1e-5), 0.0)
    t1b = jnp.where(lane3, pbeta16 - mu1 * s1, 0.0)
    s1c = jnp.tile(s1, K).reshape(1, K * 16)
    t1c = jnp.tile(t1b, K).reshape(1, K * 16)
    pb2t = jnp.tile(pb2, K).reshape(1, K * C)

    st2 = pl.pallas_call(
        _stats2_kernel,
        grid=grid,
        in_specs=[
            pl.BlockSpec((_B, K * C), lambda i: (i, 0)),
            pl.BlockSpec((_B, K * 16), lambda i: (i, 0)),
            pl.BlockSpec((_B, 16), lambda i: (i, 0)),
            pl.BlockSpec((_B, C), lambda i: (i, 0)),
            _const_spec((16, K * 16)), _const_spec((C, K * C)),
            _const_spec((1, K * 16)), _const_spec((1, K * 16)),
            _const_spec((K * 16, K * C)), _const_spec((1, K * C)),
        ],
        out_specs=pl.BlockSpec((8, K * C), lambda i: (0, 0)),
        out_shape=jax.ShapeDtypeStruct((8, K * C), f32),
    )(xkg2, ag2, a16, xq, t16, t64, s1c, t1c, wbd2, pb2t)
    mu2, var2 = _moments(st2, C)
    s2 = wg1 / jnp.sqrt(var2 + 1e-5)
    t2 = wbeta1 - mu2 * s2

    wbd1 = jnp.kron(eye16, wW1)
    w1, st3 = pl.pallas_call(
        _w1_kernel,
        grid=grid,
        in_specs=[
            pl.BlockSpec((_B, K * C), lambda i: (i, 0)),
            pl.BlockSpec((_B, K * 16), lambda i: (i, 0)),
            pl.BlockSpec((_B, 16), lambda i: (i, 0)),
            pl.BlockSpec((_B, C), lambda i: (i, 0)),
            _const_spec((16, K * 16)), _const_spec((C, K * C)),
            _const_spec((1, K * 16)), _const_spec((1, K * 16)),
            _const_spec((K * 16, K * C)), _const_spec((1, K * C)),
            _const_spec((1, K * C)), _const_spec((1, K * C)),
            _const_spec((K * C, 128)), _const_spec((1, 128)),
        ],
        out_specs=[
            pl.BlockSpec((_B, 128), lambda i: (i, 0)),
            pl.BlockSpec((8, 128), lambda i: (0, 0)),
        ],
        out_shape=[
            jax.ShapeDtypeStruct((N, 128), f32),
            jax.ShapeDtypeStruct((8, 128), f32),
        ],
    )(xkg2, ag2, a16, xq, t16, t64, s1c, t1c, wbd2, pb2t,
      jnp.tile(s2, K).reshape(1, K * C), jnp.tile(t2, K).reshape(1, K * C),
      wbd1, jnp.tile(wb1, K).reshape(1, 128))
    mu3, var3 = _moments(st3, 8)
    s3 = wg2 / jnp.sqrt(var3 + 1e-5)
    t3 = wbeta2 - mu3 * s3

    wbd0 = jnp.kron(eye16, wW2)
    out = pl.pallas_call(
        _final_kernel,
        grid=grid,
        in_specs=[
            pl.BlockSpec((_B, 128), lambda i: (i, 0)),
            pl.BlockSpec((_B, K * C), lambda i: (i, 0)),
            pl.BlockSpec((_B, K * 16), lambda i: (i, 0)),
            pl.BlockSpec((_B, 16), lambda i: (i, 0)),
            _const_spec((16, K * 16)),
            _const_spec((1, K * 16)), _const_spec((1, K * 16)),
            _const_spec((K * 16, K * C)), _const_spec((1, K * C)),
            _const_spec((1, 128)), _const_spec((1, 128)),
            _const_spec((128, 128)), _const_spec((1, 128)),
            _const_spec((128, 128)), _const_spec((128, K * C)),
            _const_spec((K * C, C)),
        ],
        out_specs=pl.BlockSpec((_B, C), lambda i: (i, 0)),
        out_shape=jax.ShapeDtypeStruct((N, C), f32),
    )(w1, xvg2, ag2, a16, t16, s1c, t1c, wbd2, pb2t,
      jnp.tile(s3, K).reshape(1, 128), jnp.tile(t3, K).reshape(1, 128),
      wbd0, jnp.tile(wb2, K).reshape(1, 128), msum, expand, reduce_m)
    return out


# speedup vs baseline: 7.3634x; 1.0003x over previous
import functools

import jax
import jax.numpy as jnp
from jax import lax
from jax.experimental import pallas as pl
from jax.experimental.pallas import tpu as pltpu
from jax.experimental.pallas import tpu_sc as plsc

N = 50000
K = 16
C = 64
FLAT = N * K
NW = 32
ROWS = 1568
RLAST = N - ROWS
NBT = ROWS * K


def _sc_mesh():
    return plsc.VectorSubcoreMesh(core_axis_name="c", subcore_axis_name="s",
                                  num_cores=2, num_subcores=16)


_SC_PARAMS = pltpu.CompilerParams(needs_layout_passes=False,
                                  use_tc_tiling_on_sc=False)


def _idx_body(edges_hbm, bnd_hbm, idx_hbm, bnd_v, edges_v, idx_v):
    w = lax.axis_index("s") * 2 + lax.axis_index("c")
    r0 = jnp.minimum(w * ROWS, RLAST)
    pltpu.sync_copy(bnd_hbm, bnd_v)
    pltpu.sync_copy(edges_hbm.at[pl.ds(r0, ROWS)], edges_v)

    def body(r, carry):
        e = edges_v[r, :]
        bg = plsc.load_gather(bnd_v, [e])
        self_vec = jnp.broadcast_to(r0 + r, (16,)).astype(jnp.int32)
        bi = plsc.load_gather(bnd_v, [self_vec])
        idx_v[r, :] = jnp.where(bg == bi, e, self_vec)
        return carry

    lax.fori_loop(0, ROWS, body, 0)
    pltpu.sync_copy(idx_v, idx_hbm.at[pl.ds(r0, ROWS)])


def _compute_idx(edges, boundary):
    return pl.kernel(
        _idx_body,
        out_type=jax.ShapeDtypeStruct((N, K), jnp.int32),
        mesh=_sc_mesh(),
        compiler_params=_SC_PARAMS,
        scratch_types=[
            pltpu.VMEM((N,), jnp.int32),
            pltpu.VMEM((ROWS, K), jnp.int32),
            pltpu.VMEM((ROWS, K), jnp.int32),
        ],
    )(edges, boundary)


GCH = 384
GNCH = 66
GLAST = NBT - GCH
NSLOT = 3


def _make_gather_body(widths):
    nt = len(widths)

    def body(*refs):
        idx_hbm = refs[0]
        tables = refs[1:1 + nt]
        outs = refs[1 + nt:1 + 2 * nt]
        idx_v = refs[1 + 2 * nt]
        bufs = refs[2 + 2 * nt:2 + 2 * nt + NSLOT * nt]
        gsems = refs[2 + 2 * nt + NSLOT * nt:2 + 2 * nt + 2 * NSLOT * nt]
        ssems = refs[2 + 2 * nt + 2 * NSLOT * nt:]
        w = lax.axis_index("s") * 2 + lax.axis_index("c")
        r0 = jnp.minimum(w * ROWS, RLAST)
        b0 = r0 * K
        pltpu.sync_copy(idx_hbm.at[pl.ds(b0, NBT)], idx_v)

        def off(c):
            return jnp.minimum(c * GCH, GLAST)

        def g_issue(c, slot):
            ix = idx_v.at[pl.ds(off(c), GCH)]
            for t in range(nt):
                pltpu.async_copy(tables[t].at[ix], bufs[NSLOT * t + slot],
                                 gsems[NSLOT * t + slot])

        def g_wait(c, slot):
            ix = idx_v.at[pl.ds(off(c), GCH)]
            for t in range(nt):
                pltpu.make_async_copy(tables[t].at[ix],
                                      bufs[NSLOT * t + slot],
                                      gsems[NSLOT * t + slot]).wait()

        def s_issue(c, slot):
            for t in range(nt):
                pltpu.async_copy(bufs[NSLOT * t + slot],
                                 outs[t].at[pl.ds(b0 + off(c), GCH)],
                                 ssems[NSLOT * t + slot])

        def s_wait(c, slot):
            for t in range(nt):
                pltpu.make_async_copy(bufs[NSLOT * t + slot],
                                      outs[t].at[pl.ds(b0 + off(c), GCH)],
                                      ssems[NSLOT * t + slot]).wait()

        g_issue(0, 0)
        g_issue(1, 1)

        def group(i, carry):
            cg = NSLOT * i
            for u in range(NSLOT):
                c = cg + u
                g_wait(c, u)
                s_issue(c, u)

                @pl.when(c >= 1)
                def _():
                    s_wait(c - 1, (u + NSLOT - 1) % NSLOT)

                @pl.when(c + 2 < GNCH)
                def _():
                    g_issue(c + 2, (u + 2) % NSLOT)
            return carry

        lax.fori_loop(0, GNCH // NSLOT, group, 0)
        s_wait(GNCH - 1, (GNCH - 1) % NSLOT)

    return body


def _gather_multi(idx_flat, tables, widths):
    return pl.kernel(
        _make_gather_body(widths),
        out_type=tuple(jax.ShapeDtypeStruct((FLAT, wd), jnp.float32)
                       for wd in widths),
        mesh=_sc_mesh(),
        compiler_params=_SC_PARAMS,
        scratch_types=[pltpu.VMEM((NBT,), jnp.int32)]
        + [pltpu.VMEM((GCH, wd), jnp.float32)
           for wd in widths for _ in range(NSLOT)]
        + [pltpu.SemaphoreType.DMA for _ in widths for _ in range(NSLOT)]
        + [pltpu.SemaphoreType.DMA for _ in widths for _ in range(NSLOT)],
    )(idx_flat, *tables)


_B1 = 2000
_B = 1000


def _qkv_kernel(x_ref, p4_ref, Wq_ref, bq_ref, Wk_ref, bk_ref, Wv_ref, bv_ref,
                pW1p_ref, xq_ref, xk_ref, xv_ref, a_ref):
    x = x_ref[...]
    xq_ref[...] = jnp.dot(x, Wq_ref[...], preferred_element_type=jnp.float32) + bq_ref[...]
    xk_ref[...] = jnp.dot(x, Wk_ref[...], preferred_element_type=jnp.float32) + bk_ref[...]
    xv_ref[...] = jnp.dot(x, Wv_ref[...], preferred_element_type=jnp.float32) + bv_ref[...]
    a_ref[...] = jnp.dot(p4_ref[...], pW1p_ref[...], preferred_element_type=jnp.float32)


def _mm(a, b):
    return jnp.dot(a, b, preferred_element_type=jnp.float32)


def _p_r(ag, a16, t16, s1, t1b, wbd2, pb2t):
    d = ag - _mm(a16, t16)
    h = jnp.maximum(d * s1 + t1b, 0.0)
    return _mm(h, wbd2) + pb2t


def _acc_stats(out_ref, s, q, width):
    @pl.when(pl.program_id(0) == 0)
    def _():
        out_ref[...] = jnp.zeros_like(out_ref)

    out_ref[...] = out_ref[...] + jnp.concatenate(
        [s[None], q[None], jnp.zeros((6, width), jnp.float32)], axis=0)


def _stats1_kernel(ag_ref, a16_ref, t16_ref, out_ref):
    d = ag_ref[...] - _mm(a16_ref[...], t16_ref[...])
    _acc_stats(out_ref, jnp.sum(d, axis=0), jnp.sum(d * d, axis=0), 256)


def _stats2_kernel(xkg_ref, ag_ref, a16_ref, xq_ref, t16_ref, t64_ref,
                   s1_ref, t1b_ref, wbd2_ref, pb2_ref, out_ref):
    pr = _p_r(ag_ref[...], a16_ref[...], t16_ref[...], s1_ref[...],
              t1b_ref[...], wbd2_ref[...], pb2_ref[...])
    wpre = xkg_ref[...] - _mm(xq_ref[...], t64_ref[...]) + pr
    _acc_stats(out_ref, jnp.sum(wpre, axis=0), jnp.sum(wpre * wpre, axis=0),
               16 * C)


def _w1_kernel(xkg_ref, ag_ref, a16_ref, xq_ref, t16_ref, t64_ref, s1_ref,
               t1b_ref, wbd2_ref, pb2_ref, s2_ref, t2_ref, wbd1_ref, wb1_ref,
               w1_ref, out_ref):
    pr = _p_r(ag_ref[...], a16_ref[...], t16_ref[...], s1_ref[...],
              t1b_ref[...], wbd2_ref[...], pb2_ref[...])
    wpre = xkg_ref[...] - _mm(xq_ref[...], t64_ref[...]) + pr
    h = jnp.maximum(wpre * s2_ref[...] + t2_ref[...], 0.0)
    w1 = _mm(h, wbd1_ref[...]) + wb1_ref[...]
    w1_ref[...] = w1
    _acc_stats(out_ref, jnp.sum(w1, axis=0), jnp.sum(w1 * w1, axis=0), 128)


def _final_kernel(w1_ref, xvg_ref, ag_ref, a16_ref, t16_ref, s1_ref, t1b_ref,
                  wbd2_ref, pb2_ref, s3_ref, t3_ref, wbd0_ref, wb2_ref,
                  msum_ref, expand_ref, reduce_ref, out_ref):
    h = jnp.maximum(w1_ref[...] * s3_ref[...] + t3_ref[...], 0.0)
    logit = _mm(h, wbd0_ref[...]) + wb2_ref[...]
    e = jnp.exp(logit)
    se = _mm(e, msum_ref[...])
    att = e / se
    attx = _mm(att, expand_ref[...])
    pr = _p_r(ag_ref[...], a16_ref[...], t16_ref[...], s1_ref[...],
              t1b_ref[...], wbd2_ref[...], pb2_ref[...])
    v = xvg_ref[...] + pr
    out_ref[...] = _mm(v * attx, reduce_ref[...])


def _const_spec(shape):
    return pl.BlockSpec(shape, lambda i: tuple(0 for _ in shape))


def _moments(stats, nch):
    m = jnp.float32(FLAT)
    mu = stats[0].reshape(K, nch).sum(0) / m
    var = jnp.maximum(stats[1].reshape(K, nch).sum(0) / m - mu * mu, 0.0)
    return mu, var


def kernel(p, x, o, edges, boundary, Wq, bq, Wk, bk, Wv, bv, pW1, pb1, pg,
           pbeta, pW2, pb2, wg1, wbeta1, wW1, wb1, wg2, wbeta2, wW2, wb2):
    f32 = jnp.float32
    p4 = jnp.pad(p, ((0, 0), (0, 1)))
    pW1p = jnp.zeros((4, 16), f32).at[:3, :3].set(pW1)

    xq, xk, xv, a16 = pl.pallas_call(
        _qkv_kernel,
        grid=(N // _B1,),
        in_specs=[
            pl.BlockSpec((_B1, C), lambda i: (i, 0)),
            pl.BlockSpec((_B1, 4), lambda i: (i, 0)),
            _const_spec((C, C)), _const_spec((1, C)),
            _const_spec((C, C)), _const_spec((1, C)),
            _const_spec((C, C)), _const_spec((1, C)),
            _const_spec((4, 16)),
        ],
        out_specs=[
            pl.BlockSpec((_B1, C), lambda i: (i, 0)),
            pl.BlockSpec((_B1, C), lambda i: (i, 0)),
            pl.BlockSpec((_B1, C), lambda i: (i, 0)),
            pl.BlockSpec((_B1, 16), lambda i: (i, 0)),
        ],
        out_shape=[
            jax.ShapeDtypeStruct((N, C), f32),
            jax.ShapeDtypeStruct((N, C), f32),
            jax.ShapeDtypeStruct((N, C), f32),
            jax.ShapeDtypeStruct((N, 16), f32),
        ],
    )(x, p4, Wq, bq.reshape(1, C), Wk, bk.reshape(1, C), Wv, bv.reshape(1, C),
      pW1p)

    idx = _compute_idx(edges.astype(jnp.int32), boundary.astype(jnp.int32))

    idx_flat = idx.reshape(FLAT)
    xkg, ag = _gather_multi(idx_flat, (xk, a16), (C, 16))
    (xvg,) = _gather_multi(idx_flat, (xv,), (C,))
    xkg2 = xkg.reshape(N, K * C)
    xvg2 = xvg.reshape(N, K * C)
    ag2 = ag.reshape(N, K * 16)

    eye16 = jnp.eye(16, dtype=f32)
    eye64 = jnp.eye(C, dtype=f32)
    eye8 = jnp.eye(8, dtype=f32)
    t16 = jnp.tile(eye16, (1, K))
    t64 = jnp.tile(eye64, (1, K))
    pW2e = jnp.zeros((16, C), f32).at[:3, :].set(pW2)
    wbd2 = jnp.kron(eye16, pW2e)
    msum = jnp.tile(eye8, (K, K))
    expand = jnp.kron(eye16, jnp.tile(eye8, (1, 8)))
    reduce_m = jnp.tile(eye64, (K, 1))

    grid = (N // _B,)

    st1 = pl.pallas_call(
        _stats1_kernel,
        grid=grid,
        in_specs=[
            pl.BlockSpec((_B, K * 16), lambda i: (i, 0)),
            pl.BlockSpec((_B, 16), lambda i: (i, 0)),
            _const_spec((16, K * 16)),
        ],
        out_specs=pl.BlockSpec((8, K * 16), lambda i: (0, 0)),
        out_shape=jax.ShapeDtypeStruct((8, K * 16), f32),
    )(ag2, a16, t16)
    mu1, var1 = _moments(st1, 16)
    lane3 = jnp.arange(16) < 3
    pg16 = jnp.zeros((16,), f32).at[:3].set(pg)
    pbeta16 = jnp.zeros((16,), f32).at[:3].set(pbeta)
    s1 = jnp.where(lane3, pg16 / jnp.sqrt(var1 + 1e-5), 0.0)
    t1b = jnp.where(lane3, pbeta16 - mu1 * s1, 0.0)
    s1c = jnp.tile(s1, K).reshape(1, K * 16)
    t1c = jnp.tile(t1b, K).reshape(1, K * 16)
    pb2t = jnp.tile(pb2, K).reshape(1, K * C)

    st2 = pl.pallas_call(
        _stats2_kernel,
        grid=grid,
        in_specs=[
            pl.BlockSpec((_B, K * C), lambda i: (i, 0)),
            pl.BlockSpec((_B, K * 16), lambda i: (i, 0)),
            pl.BlockSpec((_B, 16), lambda i: (i, 0)),
            pl.BlockSpec((_B, C), lambda i: (i, 0)),
            _const_spec((16, K * 16)), _const_spec((C, K * C)),
            _const_spec((1, K * 16)), _const_spec((1, K * 16)),
            _const_spec((K * 16, K * C)), _const_spec((1, K * C)),
        ],
        out_specs=pl.BlockSpec((8, K * C), lambda i: (0, 0)),
        out_shape=jax.ShapeDtypeStruct((8, K * C), f32),
    )(xkg2, ag2, a16, xq, t16, t64, s1c, t1c, wbd2, pb2t)
    mu2, var2 = _moments(st2, C)
    s2 = wg1 / jnp.sqrt(var2 + 1e-5)
    t2 = wbeta1 - mu2 * s2

    wbd1 = jnp.kron(eye16, wW1)
    w1, st3 = pl.pallas_call(
        _w1_kernel,
        grid=grid,
        in_specs=[
            pl.BlockSpec((_B, K * C), lambda i: (i, 0)),
            pl.BlockSpec((_B, K * 16), lambda i: (i, 0)),
            pl.BlockSpec((_B, 16), lambda i: (i, 0)),
            pl.BlockSpec((_B, C), lambda i: (i, 0)),
            _const_spec((16, K * 16)), _const_spec((C, K * C)),
            _const_spec((1, K * 16)), _const_spec((1, K * 16)),
            _const_spec((K * 16, K * C)), _const_spec((1, K * C)),
            _const_spec((1, K * C)), _const_spec((1, K * C)),
            _const_spec((K * C, 128)), _const_spec((1, 128)),
        ],
        out_specs=[
            pl.BlockSpec((_B, 128), lambda i: (i, 0)),
            pl.BlockSpec((8, 128), lambda i: (0, 0)),
        ],
        out_shape=[
            jax.ShapeDtypeStruct((N, 128), f32),
            jax.ShapeDtypeStruct((8, 128), f32),
        ],
    )(xkg2, ag2, a16, xq, t16, t64, s1c, t1c, wbd2, pb2t,
      jnp.tile(s2, K).reshape(1, K * C), jnp.tile(t2, K).reshape(1, K * C),
      wbd1, jnp.tile(wb1, K).reshape(1, 128))
    mu3, var3 = _moments(st3, 8)
    s3 = wg2 / jnp.sqrt(var3 + 1e-5)
    t3 = wbeta2 - mu3 * s3

    wbd0 = jnp.kron(eye16, wW2)
    out = pl.pallas_call(
        _final_kernel,
        grid=grid,
        in_specs=[
            pl.BlockSpec((_B, 128), lambda i: (i, 0)),
            pl.BlockSpec((_B, K * C), lambda i: (i, 0)),
            pl.BlockSpec((_B, K * 16), lambda i: (i, 0)),
            pl.BlockSpec((_B, 16), lambda i: (i, 0)),
            _const_spec((16, K * 16)),
            _const_spec((1, K * 16)), _const_spec((1, K * 16)),
            _const_spec((K * 16, K * C)), _const_spec((1, K * C)),
            _const_spec((1, 128)), _const_spec((1, 128)),
            _const_spec((128, 128)), _const_spec((1, 128)),
            _const_spec((128, 128)), _const_spec((128, K * C)),
            _const_spec((K * C, C)),
        ],
        out_specs=pl.BlockSpec((_B, C), lambda i: (i, 0)),
        out_shape=jax.ShapeDtypeStruct((N, C), f32),
    )(w1, xvg2, ag2, a16, t16, s1c, t1c, wbd2, pb2t,
      jnp.tile(s3, K).reshape(1, 128), jnp.tile(t3, K).reshape(1, 128),
      wbd0, jnp.tile(wb2, K).reshape(1, 128), msum, expand, reduce_m)
    return out
